# baseline (speedup 1.0000x reference)
import functools

import jax
import jax.numpy as jnp
from jax import lax
from jax.experimental import pallas as pl
from jax.experimental.pallas import tpu as pltpu
from jax.experimental.pallas import tpu_sc as plsc

_NC = 2
_NS = 16
_NW = _NC * _NS

_TM = 384
_NFF = 2


def _sc_mesh():
    return plsc.VectorSubcoreMesh(core_axis_name="c", subcore_axis_name="s")


def _sc_gather_rows(table, idx, chunk, actions=None):
    V, D = table.shape
    B = idx.shape[0]
    assert B % (8 * _NW) == 0
    b_per_w = B // _NW
    assert b_per_w % chunk == 0 and chunk % 8 == 0
    n_chunks = b_per_w // chunk

    out_type = [jax.ShapeDtypeStruct((B, D), table.dtype)]
    scratch = [
        pltpu.VMEM((b_per_w,), jnp.int32),
        pltpu.VMEM((chunk, D), table.dtype),
        pltpu.VMEM((chunk, D), table.dtype),
        pltpu.SemaphoreType.DMA,
        pltpu.SemaphoreType.DMA,
        pltpu.SemaphoreType.DMA,
        pltpu.SemaphoreType.DMA,
        pltpu.SemaphoreType.DMA,
    ]
    if actions is not None:
        DA = actions.shape[1]
        out_type.append(jax.ShapeDtypeStruct((B, DA), actions.dtype))
        scratch.append(pltpu.VMEM((b_per_w, DA), actions.dtype))

    def k(*refs):
        if actions is not None:
            (table_hbm, act_hbm, idx_hbm, z_out, a_out,
             idx_v, rows0, rows1, isem, g0, g1, w0, w1, asem_v) = refs
        else:
            (table_hbm, idx_hbm, z_out,
             idx_v, rows0, rows1, isem, g0, g1, w0, w1) = refs
        wid = lax.axis_index("s") * _NC + lax.axis_index("c")
        base = wid * b_per_w
        pltpu.sync_copy(idx_hbm.at[pl.ds(base, b_per_w)], idx_v)
        if actions is not None:
            a_op = pltpu.async_copy(act_hbm.at[idx_v], asem_v, isem)
        rows = [rows0, rows1]
        gsem = [g0, g1]
        wsem = [w0, w1]
        gops = [None, None]
        wops = [None, None]
        gops[0] = pltpu.async_copy(
            table_hbm.at[idx_v.at[pl.ds(0, chunk)]], rows[0], gsem[0])
        for c in range(n_chunks):
            cur = c & 1
            gops[cur].wait()
            if c + 1 < n_chunks:
                o = (c + 1) & 1
                if wops[o] is not None:
                    wops[o].wait()
                gops[o] = pltpu.async_copy(
                    table_hbm.at[idx_v.at[pl.ds((c + 1) * chunk, chunk)]],
                    rows[o], gsem[o])
            wops[cur] = pltpu.async_copy(
                rows[cur], z_out.at[pl.ds(base + c * chunk, chunk)], wsem[cur])
        if n_chunks >= 2:
            wops[(n_chunks - 2) & 1].wait()
        wops[(n_chunks - 1) & 1].wait()
        if actions is not None:
            a_op.wait()
            pltpu.sync_copy(asem_v, a_out.at[pl.ds(base, b_per_w)])

    fn = pl.kernel(k, out_type=tuple(out_type) if actions is not None
                   else out_type[0],
                   mesh=_sc_mesh(), scratch_types=scratch)
    if actions is not None:
        return fn(table, actions, idx)
    return fn(table, idx)


def _tc_grouped_mlp(block_pol, z_sorted, a_sorted, W1, b1r, W2, b2):
    NPAD, DM = z_sorted.shape
    P, DK, DF = W1.shape
    NB = NPAD // _TM
    FB = DF // _NFF

    def _fe(b, f):
        return jnp.where(b % 2 == 0, f, _NFF - 1 - f)

    def body(bp_ref, z_ref, a_ref, w1_ref, w2_ref, b1_ref, b2_ref, o_ref):
        b = pl.program_id(0)
        f = pl.program_id(1)
        fe = _fe(b, f)
        pol = bp_ref[b]
        zb = z_ref[...].astype(jnp.bfloat16)
        ab = a_ref[...].astype(jnp.bfloat16)
        w1 = w1_ref[0].astype(jnp.bfloat16)
        h = jnp.dot(zb, w1[:DM], preferred_element_type=jnp.float32)
        h = h + jnp.dot(ab, w1[DK - 128:], preferred_element_type=jnp.float32)
        h = h + b1_ref[pol, fe][None, :]
        h = jnp.maximum(h, 0.0).astype(jnp.bfloat16)
        part = jnp.dot(h, w2_ref[0].astype(jnp.bfloat16),
                       preferred_element_type=jnp.float32)

        @pl.when(f == 0)
        def _():
            o_ref[...] = part + b2_ref[pol][None, :]

        @pl.when(f != 0)
        def _():
            o_ref[...] = o_ref[...] + part

    grid_spec = pltpu.PrefetchScalarGridSpec(
        num_scalar_prefetch=1,
        grid=(NB, _NFF),
        in_specs=[
            pl.BlockSpec((_TM, DM), lambda b, f, bp: (b, 0)),
            pl.BlockSpec((_TM, 128), lambda b, f, bp: (b, 0)),
            pl.BlockSpec((1, DK, FB), lambda b, f, bp: (bp[b], 0, _fe(b, f))),
            pl.BlockSpec((1, FB, DM), lambda b, f, bp: (bp[b], _fe(b, f), 0)),
            pl.BlockSpec((P, _NFF, FB), lambda b, f, bp: (0, 0, 0)),
            pl.BlockSpec((P, DM), lambda b, f, bp: (0, 0)),
        ],
        out_specs=pl.BlockSpec((_TM, DM), lambda b, f, bp: (b, 0)),
    )
    return pl.pallas_call(
        body,
        grid_spec=grid_spec,
        out_shape=jax.ShapeDtypeStruct((NPAD, DM), jnp.float32),
        compiler_params=pltpu.CompilerParams(
            dimension_semantics=("arbitrary", "arbitrary"),
        ),
    )(block_pol, z_sorted, a_sorted, W1, W2, b1r, b2)


def _route(pi, P, N, NB):
    TM = _TM
    NPAD = NB * TM
    sort_idx = jnp.argsort(pi, stable=True).astype(jnp.int32)
    counts = jnp.bincount(pi, length=P).astype(jnp.int32)
    csum = jnp.cumsum(counts)
    group_start = jnp.concatenate([jnp.zeros((1,), jnp.int32),
                                   csum[:-1].astype(jnp.int32)])
    nblk = (counts + TM - 1) // TM
    bsum = jnp.cumsum(nblk)
    blk_start = jnp.concatenate([jnp.zeros((1,), jnp.int32),
                                 bsum[:-1].astype(jnp.int32)])
    padded_start = blk_start * TM

    r = jnp.arange(NPAD, dtype=jnp.int32)
    g_r = (jnp.searchsorted(padded_start, r, side="right") - 1).astype(jnp.int32)
    off = r - padded_start[g_r]
    live = off < counts[g_r]
    spos = group_start[g_r] + jnp.minimum(off, jnp.maximum(counts[g_r] - 1, 0))
    row_src = jnp.where(live, sort_idx[spos], 0).astype(jnp.int32)

    block_pol = g_r[::TM]

    s = jnp.arange(N, dtype=jnp.int32)
    gs = pi[sort_idx]
    pos_val = padded_start[gs] + (s - group_start[gs])
    pos = jnp.zeros((N,), jnp.int32).at[sort_idx].set(pos_val.astype(jnp.int32))
    return row_src, block_pol, pos


def kernel(latents, policy_indices, actions, W1, b1, W2, b2):
    N, DM = latents.shape
    DA = actions.shape[1]
    P, DK, DF = W1.shape
    FB = DF // _NFF

    NB = N // _TM + P
    while (NB * _TM) % (8 * _NW) != 0:
        NB += 1

    pi = policy_indices.astype(jnp.int32)
    row_src, block_pol, pos = _route(pi, P, N, NB)

    a_pad = jnp.pad(actions, ((0, 0), (128 - DA, 0)))
    z_sorted = _sc_gather_rows(latents, row_src, chunk=40)
    a_sorted = _sc_gather_rows(a_pad, row_src, chunk=NB * _TM // _NW)

    b1r = b1.reshape(P, _NFF, FB)
    y_sorted = _tc_grouped_mlp(block_pol, z_sorted, a_sorted, W1, b1r, W2, b2)

    out = _sc_gather_rows(y_sorted, pos, chunk=32)
    return out


# speedup vs baseline: 1.0169x; 1.0169x over previous
import functools

import jax
import jax.numpy as jnp
from jax import lax
from jax.experimental import pallas as pl
from jax.experimental.pallas import tpu as pltpu
from jax.experimental.pallas import tpu_sc as plsc

_NC = 2
_NS = 16
_NW = _NC * _NS

_TM = 384
_NFF = 2


def _sc_mesh():
    return plsc.VectorSubcoreMesh(core_axis_name="c", subcore_axis_name="s")


def _sc_gather_rows(table, idx, chunk):
    V, D = table.shape
    B = idx.shape[0]
    assert B % (8 * _NW) == 0
    b_per_w = B // _NW
    assert b_per_w % chunk == 0 and chunk % 8 == 0
    n_chunks = b_per_w // chunk

    @functools.partial(
        pl.kernel,
        out_type=jax.ShapeDtypeStruct((B, D), table.dtype),
        mesh=_sc_mesh(),
        scratch_types=[
            pltpu.VMEM((b_per_w,), jnp.int32),
            pltpu.VMEM((chunk, D), table.dtype),
            pltpu.VMEM((chunk, D), table.dtype),
            pltpu.SemaphoreType.DMA,
            pltpu.SemaphoreType.DMA,
            pltpu.SemaphoreType.DMA,
            pltpu.SemaphoreType.DMA,
        ],
    )
    def k(table_hbm, idx_hbm, out_hbm, idx_v, rows0, rows1, g0, g1, w0, w1):
        wid = lax.axis_index("s") * _NC + lax.axis_index("c")
        base = wid * b_per_w
        pltpu.sync_copy(idx_hbm.at[pl.ds(base, b_per_w)], idx_v)
        rows = [rows0, rows1]
        gsem = [g0, g1]
        wsem = [w0, w1]
        gops = [None, None]
        wops = [None, None]
        gops[0] = pltpu.async_copy(
            table_hbm.at[idx_v.at[pl.ds(0, chunk)]], rows[0], gsem[0])
        for c in range(n_chunks):
            cur = c & 1
            gops[cur].wait()
            if c + 1 < n_chunks:
                o = (c + 1) & 1
                if wops[o] is not None:
                    wops[o].wait()
                gops[o] = pltpu.async_copy(
                    table_hbm.at[idx_v.at[pl.ds((c + 1) * chunk, chunk)]],
                    rows[o], gsem[o])
            wops[cur] = pltpu.async_copy(
                rows[cur], out_hbm.at[pl.ds(base + c * chunk, chunk)], wsem[cur])
        if n_chunks >= 2:
            wops[(n_chunks - 2) & 1].wait()
        wops[(n_chunks - 1) & 1].wait()

    return k(table, idx)


def _tc_prep(latents, a_pad):
    N, DM = latents.shape
    DX = DM + 128
    RB = 512

    def body(z_ref, a_ref, x_ref):
        x_ref[:, :DM] = z_ref[...]
        x_ref[:, DM:] = a_ref[...]

    return pl.pallas_call(
        body,
        grid=(N // RB,),
        in_specs=[pl.BlockSpec((RB, DM), lambda i: (i, 0)),
                  pl.BlockSpec((RB, 128), lambda i: (i, 0))],
        out_specs=pl.BlockSpec((RB, DX), lambda i: (i, 0)),
        out_shape=jax.ShapeDtypeStruct((N, DX), jnp.float32),
        compiler_params=pltpu.CompilerParams(
            dimension_semantics=("arbitrary",),
        ),
    )(latents, a_pad)


def _tc_grouped_mlp(block_pol, x_sorted, W1, b1r, W2, b2):
    NPAD, DX = x_sorted.shape
    DM = DX - 128
    P, DK, DF = W1.shape
    NB = NPAD // _TM
    FB = DF // _NFF

    def _fe(b, f):
        return jnp.where(b % 2 == 0, f, _NFF - 1 - f)

    def body(bp_ref, x_ref, w1_ref, w2_ref, b1_ref, b2_ref, o_ref):
        b = pl.program_id(0)
        f = pl.program_id(1)
        fe = _fe(b, f)
        pol = bp_ref[b]
        zb = x_ref[:, :DM].astype(jnp.bfloat16)
        ab = x_ref[:, DM:].astype(jnp.bfloat16)
        w1 = w1_ref[0]
        h = jnp.dot(zb, w1[:DM], preferred_element_type=jnp.float32)
        h = h + jnp.dot(ab, w1[DK - 128:], preferred_element_type=jnp.float32)
        h = h + b1_ref[pol, fe][None, :]
        h = jnp.maximum(h, 0.0).astype(jnp.bfloat16)
        part = jnp.dot(h, w2_ref[0], preferred_element_type=jnp.float32)

        @pl.when(f == 0)
        def _():
            o_ref[...] = part + b2_ref[pol][None, :]

        @pl.when(f != 0)
        def _():
            o_ref[...] = o_ref[...] + part

    grid_spec = pltpu.PrefetchScalarGridSpec(
        num_scalar_prefetch=1,
        grid=(NB, _NFF),
        in_specs=[
            pl.BlockSpec((_TM, DX), lambda b, f, bp: (b, 0)),
            pl.BlockSpec((1, DK, FB), lambda b, f, bp: (bp[b], 0, _fe(b, f))),
            pl.BlockSpec((1, FB, DM), lambda b, f, bp: (bp[b], _fe(b, f), 0)),
            pl.BlockSpec((P, _NFF, FB), lambda b, f, bp: (0, 0, 0)),
            pl.BlockSpec((P, DM), lambda b, f, bp: (0, 0)),
        ],
        out_specs=pl.BlockSpec((_TM, DM), lambda b, f, bp: (b, 0)),
    )
    return pl.pallas_call(
        body,
        grid_spec=grid_spec,
        out_shape=jax.ShapeDtypeStruct((NPAD, DM), jnp.float32),
        compiler_params=pltpu.CompilerParams(
            dimension_semantics=("arbitrary", "arbitrary"),
        ),
    )(block_pol, x_sorted, W1, W2, b1r, b2)


def _route(pi, P, N, NB):
    TM = _TM
    NPAD = NB * TM
    sort_idx = jnp.argsort(pi, stable=True).astype(jnp.int32)
    counts = jnp.bincount(pi, length=P).astype(jnp.int32)
    csum = jnp.cumsum(counts)
    group_start = jnp.concatenate([jnp.zeros((1,), jnp.int32),
                                   csum[:-1].astype(jnp.int32)])
    nblk = (counts + TM - 1) // TM
    bsum = jnp.cumsum(nblk)
    blk_start = jnp.concatenate([jnp.zeros((1,), jnp.int32),
                                 bsum[:-1].astype(jnp.int32)])
    padded_start = blk_start * TM

    r = jnp.arange(NPAD, dtype=jnp.int32)
    g_r = (jnp.searchsorted(padded_start, r, side="right") - 1).astype(jnp.int32)
    off = r - padded_start[g_r]
    live = off < counts[g_r]
    spos = group_start[g_r] + jnp.minimum(off, jnp.maximum(counts[g_r] - 1, 0))
    row_src = jnp.where(live, sort_idx[spos], 0).astype(jnp.int32)

    block_pol = g_r[::TM]

    s = jnp.arange(N, dtype=jnp.int32)
    gs = pi[sort_idx]
    pos_val = padded_start[gs] + (s - group_start[gs])
    pos = jnp.zeros((N,), jnp.int32).at[sort_idx].set(pos_val.astype(jnp.int32))
    return row_src, block_pol, pos


def kernel(latents, policy_indices, actions, W1, b1, W2, b2):
    N, DM = latents.shape
    DA = actions.shape[1]
    P, DK, DF = W1.shape
    FB = DF // _NFF

    NB = N // _TM + P
    while (NB * _TM) % (8 * _NW) != 0:
        NB += 1

    pi = policy_indices.astype(jnp.int32)
    row_src, block_pol, pos = _route(pi, P, N, NB)

    a_pad = jnp.pad(actions, ((0, 0), (128 - DA, 0)))
    x_lin = _tc_prep(latents, a_pad)
    x_sorted = _sc_gather_rows(x_lin, row_src, chunk=40)

    b1r = b1.reshape(P, _NFF, FB)
    W1b = W1.astype(jnp.bfloat16)
    W2b = W2.astype(jnp.bfloat16)
    y_sorted = _tc_grouped_mlp(block_pol, x_sorted, W1b, b1r, W2b, b2)

    out = _sc_gather_rows(y_sorted, pos, chunk=32)
    return out


# speedup vs baseline: 1.3206x; 1.2986x over previous
import functools

import jax
import jax.numpy as jnp
from jax import lax
from jax.experimental import pallas as pl
from jax.experimental.pallas import tpu as pltpu
from jax.experimental.pallas import tpu_sc as plsc

_NC = 2
_NS = 16
_NW = _NC * _NS

_TM = 384
_NFF = 2


def _sc_mesh():
    return plsc.VectorSubcoreMesh(core_axis_name="c", subcore_axis_name="s")


def _sc_gather_rows(table, idx, chunk):
    V, D = table.shape
    B = idx.shape[0]
    assert B % (8 * _NW) == 0
    b_per_w = B // _NW
    assert b_per_w % chunk == 0 and chunk % 8 == 0
    n_chunks = b_per_w // chunk

    @functools.partial(
        pl.kernel,
        out_type=jax.ShapeDtypeStruct((B, D), table.dtype),
        mesh=_sc_mesh(),
        scratch_types=[
            pltpu.VMEM((b_per_w,), jnp.int32),
            pltpu.VMEM((chunk, D), table.dtype),
            pltpu.VMEM((chunk, D), table.dtype),
            pltpu.SemaphoreType.DMA,
            pltpu.SemaphoreType.DMA,
            pltpu.SemaphoreType.DMA,
            pltpu.SemaphoreType.DMA,
        ],
    )
    def k(table_hbm, idx_hbm, out_hbm, idx_v, rows0, rows1, g0, g1, w0, w1):
        wid = lax.axis_index("s") * _NC + lax.axis_index("c")
        base = wid * b_per_w
        pltpu.sync_copy(idx_hbm.at[pl.ds(base, b_per_w)], idx_v)
        rows = [rows0, rows1]
        gsem = [g0, g1]
        wsem = [w0, w1]
        gops = [None, None]
        wops = [None, None]
        gops[0] = pltpu.async_copy(
            table_hbm.at[idx_v.at[pl.ds(0, chunk)]], rows[0], gsem[0])
        for c in range(n_chunks):
            cur = c & 1
            gops[cur].wait()
            if c + 1 < n_chunks:
                o = (c + 1) & 1
                if wops[o] is not None:
                    wops[o].wait()
                gops[o] = pltpu.async_copy(
                    table_hbm.at[idx_v.at[pl.ds((c + 1) * chunk, chunk)]],
                    rows[o], gsem[o])
            wops[cur] = pltpu.async_copy(
                rows[cur], out_hbm.at[pl.ds(base + c * chunk, chunk)], wsem[cur])
        if n_chunks >= 2:
            wops[(n_chunks - 2) & 1].wait()
        wops[(n_chunks - 1) & 1].wait()

    return k(table, idx)


def _tc_prep(latents, a_pad):
    N, DM = latents.shape
    DX = DM + 128
    RB = 512

    def body(z_ref, a_ref, x_ref):
        x_ref[:, :DM] = z_ref[...]
        x_ref[:, DM:] = a_ref[...]

    return pl.pallas_call(
        body,
        grid=(N // RB,),
        in_specs=[pl.BlockSpec((RB, DM), lambda i: (i, 0)),
                  pl.BlockSpec((RB, 128), lambda i: (i, 0))],
        out_specs=pl.BlockSpec((RB, DX), lambda i: (i, 0)),
        out_shape=jax.ShapeDtypeStruct((N, DX), jnp.float32),
        compiler_params=pltpu.CompilerParams(
            dimension_semantics=("arbitrary",),
        ),
    )(latents, a_pad)


def _tc_grouped_mlp(block_pol, x_sorted, W1, b1r, W2, b2):
    NPAD, DX = x_sorted.shape
    DM = DX - 128
    P, DK, DF = W1.shape
    NB = NPAD // _TM
    FB = DF // _NFF

    def _fe(b, f):
        return jnp.where(b % 2 == 0, f, _NFF - 1 - f)

    def body(bp_ref, x_ref, w1_ref, w2_ref, b1_ref, b2_ref, o_ref):
        b = pl.program_id(0)
        f = pl.program_id(1)
        fe = _fe(b, f)
        pol = bp_ref[b]
        zb = x_ref[:, :DM].astype(jnp.bfloat16)
        ab = x_ref[:, DM:].astype(jnp.bfloat16)
        w1 = w1_ref[0]
        h = jnp.dot(zb, w1[:DM], preferred_element_type=jnp.float32)
        h = h + jnp.dot(ab, w1[DK - 128:], preferred_element_type=jnp.float32)
        h = h + b1_ref[pol, fe][None, :]
        h = jnp.maximum(h, 0.0).astype(jnp.bfloat16)
        part = jnp.dot(h, w2_ref[0], preferred_element_type=jnp.float32)

        @pl.when(f == 0)
        def _():
            o_ref[...] = part + b2_ref[pol][None, :]

        @pl.when(f != 0)
        def _():
            o_ref[...] = o_ref[...] + part

    grid_spec = pltpu.PrefetchScalarGridSpec(
        num_scalar_prefetch=1,
        grid=(NB, _NFF),
        in_specs=[
            pl.BlockSpec((_TM, DX), lambda b, f, bp: (b, 0)),
            pl.BlockSpec((1, DK, FB), lambda b, f, bp: (bp[b], 0, _fe(b, f))),
            pl.BlockSpec((1, FB, DM), lambda b, f, bp: (bp[b], _fe(b, f), 0)),
            pl.BlockSpec((P, _NFF, FB), lambda b, f, bp: (0, 0, 0)),
            pl.BlockSpec((P, DM), lambda b, f, bp: (0, 0)),
        ],
        out_specs=pl.BlockSpec((_TM, DM), lambda b, f, bp: (b, 0)),
    )
    return pl.pallas_call(
        body,
        grid_spec=grid_spec,
        out_shape=jax.ShapeDtypeStruct((NPAD, DM), jnp.float32),
        compiler_params=pltpu.CompilerParams(
            dimension_semantics=("arbitrary", "arbitrary"),
        ),
    )(block_pol, x_sorted, W1, W2, b1r, b2)


def _route(pi, P, N, NB):
    TM = _TM
    NPAD = NB * TM
    sort_idx = jnp.argsort(pi, stable=True).astype(jnp.int32)
    counts = jnp.bincount(pi, length=P).astype(jnp.int32)
    csum = jnp.cumsum(counts)
    group_start = jnp.concatenate([jnp.zeros((1,), jnp.int32),
                                   csum[:-1].astype(jnp.int32)])
    nblk = (counts + TM - 1) // TM
    bsum = jnp.cumsum(nblk)
    blk_start = jnp.concatenate([jnp.zeros((1,), jnp.int32),
                                 bsum[:-1].astype(jnp.int32)])
    padded_start = blk_start * TM

    r = jnp.arange(NPAD, dtype=jnp.int32)
    g_r = (jnp.searchsorted(padded_start, r, side="right") - 1).astype(jnp.int32)
    off = r - padded_start[g_r]
    live = off < counts[g_r]
    spos = group_start[g_r] + jnp.minimum(off, jnp.maximum(counts[g_r] - 1, 0))
    row_src = jnp.where(live, sort_idx[spos], r % N).astype(jnp.int32)

    block_pol = g_r[::TM]

    s = jnp.arange(N, dtype=jnp.int32)
    gs = pi[sort_idx]
    pos_val = padded_start[gs] + (s - group_start[gs])
    pos = jnp.zeros((N,), jnp.int32).at[sort_idx].set(pos_val.astype(jnp.int32))
    return row_src, block_pol, pos


def kernel(latents, policy_indices, actions, W1, b1, W2, b2):
    N, DM = latents.shape
    DA = actions.shape[1]
    P, DK, DF = W1.shape
    FB = DF // _NFF

    NB = N // _TM + P
    while (NB * _TM) % (8 * _NW) != 0:
        NB += 1

    pi = policy_indices.astype(jnp.int32)
    row_src, block_pol, pos = _route(pi, P, N, NB)

    a_pad = jnp.pad(actions, ((0, 0), (128 - DA, 0)))
    x_lin = _tc_prep(latents, a_pad)
    x_sorted = _sc_gather_rows(x_lin, row_src, chunk=40)

    b1r = b1.reshape(P, _NFF, FB)
    W1b = W1.astype(jnp.bfloat16)
    W2b = W2.astype(jnp.bfloat16)
    y_sorted = _tc_grouped_mlp(block_pol, x_sorted, W1b, b1r, W2b, b2)

    out = _sc_gather_rows(y_sorted, pos, chunk=32)
    return out


# speedup vs baseline: 1.3607x; 1.0303x over previous
import functools

import jax
import jax.numpy as jnp
from jax import lax
from jax.experimental import pallas as pl
from jax.experimental.pallas import tpu as pltpu
from jax.experimental.pallas import tpu_sc as plsc

_NC = 2
_NS = 16
_NW = _NC * _NS

_TM = 384
_NFF = 1


def _sc_mesh():
    return plsc.VectorSubcoreMesh(core_axis_name="c", subcore_axis_name="s")


def _sc_gather_rows(table, idx, chunk):
    V, D = table.shape
    B = idx.shape[0]
    assert B % (8 * _NW) == 0
    b_per_w = B // _NW
    assert b_per_w % chunk == 0 and chunk % 8 == 0
    n_chunks = b_per_w // chunk

    @functools.partial(
        pl.kernel,
        out_type=jax.ShapeDtypeStruct((B, D), table.dtype),
        mesh=_sc_mesh(),
        scratch_types=[
            pltpu.VMEM((b_per_w,), jnp.int32),
            pltpu.VMEM((chunk, D), table.dtype),
            pltpu.VMEM((chunk, D), table.dtype),
            pltpu.SemaphoreType.DMA,
            pltpu.SemaphoreType.DMA,
            pltpu.SemaphoreType.DMA,
            pltpu.SemaphoreType.DMA,
        ],
    )
    def k(table_hbm, idx_hbm, out_hbm, idx_v, rows0, rows1, g0, g1, w0, w1):
        wid = lax.axis_index("s") * _NC + lax.axis_index("c")
        base = wid * b_per_w
        pltpu.sync_copy(idx_hbm.at[pl.ds(base, b_per_w)], idx_v)
        rows = [rows0, rows1]
        gsem = [g0, g1]
        wsem = [w0, w1]
        gops = [None, None]
        wops = [None, None]
        gops[0] = pltpu.async_copy(
            table_hbm.at[idx_v.at[pl.ds(0, chunk)]], rows[0], gsem[0])
        for c in range(n_chunks):
            cur = c & 1
            gops[cur].wait()
            if c + 1 < n_chunks:
                o = (c + 1) & 1
                if wops[o] is not None:
                    wops[o].wait()
                gops[o] = pltpu.async_copy(
                    table_hbm.at[idx_v.at[pl.ds((c + 1) * chunk, chunk)]],
                    rows[o], gsem[o])
            wops[cur] = pltpu.async_copy(
                rows[cur], out_hbm.at[pl.ds(base + c * chunk, chunk)], wsem[cur])
        if n_chunks >= 2:
            wops[(n_chunks - 2) & 1].wait()
        wops[(n_chunks - 1) & 1].wait()

    return k(table, idx)


def _tc_prep(latents, a_pad):
    N, DM = latents.shape
    DX = DM + 128
    RB = 512

    def body(z_ref, a_ref, x_ref):
        x_ref[:, :DM] = z_ref[...]
        x_ref[:, DM:] = a_ref[...]

    return pl.pallas_call(
        body,
        grid=(N // RB,),
        in_specs=[pl.BlockSpec((RB, DM), lambda i: (i, 0)),
                  pl.BlockSpec((RB, 128), lambda i: (i, 0))],
        out_specs=pl.BlockSpec((RB, DX), lambda i: (i, 0)),
        out_shape=jax.ShapeDtypeStruct((N, DX), jnp.float32),
        compiler_params=pltpu.CompilerParams(
            dimension_semantics=("arbitrary",),
        ),
    )(latents, a_pad)


def _tc_grouped_mlp(block_pol, x_sorted, W1, b1r, W2, b2):
    NPAD, DX = x_sorted.shape
    DM = DX - 128
    P, DK, DF = W1.shape
    NB = NPAD // _TM
    FB = DF // _NFF

    def _fe(b, f):
        return jnp.where(b % 2 == 0, f, _NFF - 1 - f)

    def body(bp_ref, x_ref, w1_ref, w2_ref, b1_ref, b2_ref, o_ref):
        b = pl.program_id(0)
        f = pl.program_id(1)
        fe = _fe(b, f)
        pol = bp_ref[b]
        zb = x_ref[:, :DM].astype(jnp.bfloat16)
        ab = x_ref[:, DM:].astype(jnp.bfloat16)
        w1 = w1_ref[0]
        h = jnp.dot(zb, w1[:DM], preferred_element_type=jnp.float32)
        h = h + jnp.dot(ab, w1[DK - 128:], preferred_element_type=jnp.float32)
        h = h + b1_ref[pol, fe][None, :]
        h = jnp.maximum(h, 0.0).astype(jnp.bfloat16)
        part = jnp.dot(h, w2_ref[0], preferred_element_type=jnp.float32)

        @pl.when(f == 0)
        def _():
            o_ref[...] = part + b2_ref[pol][None, :]

        @pl.when(f != 0)
        def _():
            o_ref[...] = o_ref[...] + part

    grid_spec = pltpu.PrefetchScalarGridSpec(
        num_scalar_prefetch=1,
        grid=(NB, _NFF),
        in_specs=[
            pl.BlockSpec((_TM, DX), lambda b, f, bp: (b, 0)),
            pl.BlockSpec((1, DK, FB), lambda b, f, bp: (bp[b], 0, _fe(b, f))),
            pl.BlockSpec((1, FB, DM), lambda b, f, bp: (bp[b], _fe(b, f), 0)),
            pl.BlockSpec((P, _NFF, FB), lambda b, f, bp: (0, 0, 0)),
            pl.BlockSpec((P, DM), lambda b, f, bp: (0, 0)),
        ],
        out_specs=pl.BlockSpec((_TM, DM), lambda b, f, bp: (b, 0)),
    )
    return pl.pallas_call(
        body,
        grid_spec=grid_spec,
        out_shape=jax.ShapeDtypeStruct((NPAD, DM), jnp.float32),
        compiler_params=pltpu.CompilerParams(
            dimension_semantics=("arbitrary", "arbitrary"),
        ),
    )(block_pol, x_sorted, W1, W2, b1r, b2)


def _route(pi, P, N, NB):
    TM = _TM
    NPAD = NB * TM
    sort_idx = jnp.argsort(pi, stable=True).astype(jnp.int32)
    counts = jnp.bincount(pi, length=P).astype(jnp.int32)
    csum = jnp.cumsum(counts)
    group_start = jnp.concatenate([jnp.zeros((1,), jnp.int32),
                                   csum[:-1].astype(jnp.int32)])
    nblk = (counts + TM - 1) // TM
    bsum = jnp.cumsum(nblk)
    blk_start = jnp.concatenate([jnp.zeros((1,), jnp.int32),
                                 bsum[:-1].astype(jnp.int32)])
    padded_start = blk_start * TM

    r = jnp.arange(NPAD, dtype=jnp.int32)
    g_r = (jnp.searchsorted(padded_start, r, side="right") - 1).astype(jnp.int32)
    off = r - padded_start[g_r]
    live = off < counts[g_r]
    spos = group_start[g_r] + jnp.minimum(off, jnp.maximum(counts[g_r] - 1, 0))
    row_src = jnp.where(live, sort_idx[spos], r % N).astype(jnp.int32)

    block_pol = g_r[::TM]

    s = jnp.arange(N, dtype=jnp.int32)
    gs = pi[sort_idx]
    pos_val = padded_start[gs] + (s - group_start[gs])
    pos = jnp.zeros((N,), jnp.int32).at[sort_idx].set(pos_val.astype(jnp.int32))
    return row_src, block_pol, pos


def kernel(latents, policy_indices, actions, W1, b1, W2, b2):
    N, DM = latents.shape
    DA = actions.shape[1]
    P, DK, DF = W1.shape
    FB = DF // _NFF

    NB = N // _TM + P
    while (NB * _TM) % (8 * _NW) != 0:
        NB += 1

    pi = policy_indices.astype(jnp.int32)
    row_src, block_pol, pos = _route(pi, P, N, NB)

    a_pad = jnp.pad(actions, ((0, 0), (128 - DA, 0)))
    x_lin = _tc_prep(latents, a_pad)
    x_sorted = _sc_gather_rows(x_lin, row_src, chunk=40)

    b1r = b1.reshape(P, _NFF, FB)
    W1b = W1.astype(jnp.bfloat16)
    W2b = W2.astype(jnp.bfloat16)
    y_sorted = _tc_grouped_mlp(block_pol, x_sorted, W1b, b1r, W2b, b2)

    out = _sc_gather_rows(y_sorted, pos, chunk=32)
    return out


# speedup vs baseline: 1.5618x; 1.1478x over previous
import functools

import jax
import jax.numpy as jnp
from jax import lax
from jax.experimental import pallas as pl
from jax.experimental.pallas import tpu as pltpu
from jax.experimental.pallas import tpu_sc as plsc

_NC = 2
_NS = 16
_NW = _NC * _NS

_TM = 384
_NFF = 1


def _sc_mesh():
    return plsc.VectorSubcoreMesh(core_axis_name="c", subcore_axis_name="s")


def _sc_gather_rows(table, idx, chunk):
    V, D = table.shape
    B = idx.shape[0]
    assert B % (8 * _NW) == 0
    b_per_w = B // _NW
    assert b_per_w % chunk == 0 and chunk % 8 == 0
    n_chunks = b_per_w // chunk

    @functools.partial(
        pl.kernel,
        out_type=jax.ShapeDtypeStruct((B, D), table.dtype),
        mesh=_sc_mesh(),
        scratch_types=[
            pltpu.VMEM((b_per_w,), jnp.int32),
            pltpu.VMEM((chunk, D), table.dtype),
            pltpu.VMEM((chunk, D), table.dtype),
            pltpu.SemaphoreType.DMA,
            pltpu.SemaphoreType.DMA,
            pltpu.SemaphoreType.DMA,
            pltpu.SemaphoreType.DMA,
        ],
    )
    def k(table_hbm, idx_hbm, out_hbm, idx_v, rows0, rows1, g0, g1, w0, w1):
        wid = lax.axis_index("s") * _NC + lax.axis_index("c")
        base = wid * b_per_w
        pltpu.sync_copy(idx_hbm.at[pl.ds(base, b_per_w)], idx_v)
        rows = [rows0, rows1]
        gsem = [g0, g1]
        wsem = [w0, w1]
        gops = [None, None]
        wops = [None, None]
        gops[0] = pltpu.async_copy(
            table_hbm.at[idx_v.at[pl.ds(0, chunk)]], rows[0], gsem[0])
        for c in range(n_chunks):
            cur = c & 1
            gops[cur].wait()
            if c + 1 < n_chunks:
                o = (c + 1) & 1
                if wops[o] is not None:
                    wops[o].wait()
                gops[o] = pltpu.async_copy(
                    table_hbm.at[idx_v.at[pl.ds((c + 1) * chunk, chunk)]],
                    rows[o], gsem[o])
            wops[cur] = pltpu.async_copy(
                rows[cur], out_hbm.at[pl.ds(base + c * chunk, chunk)], wsem[cur])
        if n_chunks >= 2:
            wops[(n_chunks - 2) & 1].wait()
        wops[(n_chunks - 1) & 1].wait()

    return k(table, idx)


def _tc_prep(latents, a_pad):
    N, DM = latents.shape
    DX = DM + 128
    RB = 512

    def body(z_ref, a_ref, x_ref):
        x_ref[:, :DM] = z_ref[...]
        x_ref[:, DM:] = a_ref[...]

    return pl.pallas_call(
        body,
        grid=(N // RB,),
        in_specs=[pl.BlockSpec((RB, DM), lambda i: (i, 0)),
                  pl.BlockSpec((RB, 128), lambda i: (i, 0))],
        out_specs=pl.BlockSpec((RB, DX), lambda i: (i, 0)),
        out_shape=jax.ShapeDtypeStruct((N, DX), jnp.float32),
        compiler_params=pltpu.CompilerParams(
            dimension_semantics=("arbitrary",),
        ),
    )(latents, a_pad)


def _tc_grouped_mlp(block_pol, x_sorted, W1, b1r, W2, b2):
    NPAD, DX = x_sorted.shape
    DM = DX - 128
    P, DK, DF = W1.shape
    NB = NPAD // _TM
    FB = DF // _NFF

    def _fe(b, f):
        return jnp.where(b % 2 == 0, f, _NFF - 1 - f)

    def body(bp_ref, x_ref, w1_ref, w2_ref, b1_ref, b2_ref, o_ref):
        b = pl.program_id(0)
        f = pl.program_id(1)
        fe = _fe(b, f)
        pol = bp_ref[b]
        zb = x_ref[:, :DM].astype(jnp.bfloat16)
        ab = x_ref[:, DM:].astype(jnp.bfloat16)
        w1 = w1_ref[0]
        h = jnp.dot(zb, w1[:DM], preferred_element_type=jnp.float32)
        h = h + jnp.dot(ab, w1[DK - 128:], preferred_element_type=jnp.float32)
        h = h + b1_ref[pol, fe][None, :]
        h = jnp.maximum(h, 0.0).astype(jnp.bfloat16)
        part = jnp.dot(h, w2_ref[0], preferred_element_type=jnp.float32)

        @pl.when(f == 0)
        def _():
            o_ref[...] = part + b2_ref[pol][None, :]

        @pl.when(f != 0)
        def _():
            o_ref[...] = o_ref[...] + part

    grid_spec = pltpu.PrefetchScalarGridSpec(
        num_scalar_prefetch=1,
        grid=(NB, _NFF),
        in_specs=[
            pl.BlockSpec((_TM, DX), lambda b, f, bp: (b, 0)),
            pl.BlockSpec((1, DK, FB), lambda b, f, bp: (bp[b], 0, _fe(b, f))),
            pl.BlockSpec((1, FB, DM), lambda b, f, bp: (bp[b], _fe(b, f), 0)),
            pl.BlockSpec((P, _NFF, FB), lambda b, f, bp: (0, 0, 0)),
            pl.BlockSpec((P, DM), lambda b, f, bp: (0, 0)),
        ],
        out_specs=pl.BlockSpec((_TM, DM), lambda b, f, bp: (b, 0)),
    )
    return pl.pallas_call(
        body,
        grid_spec=grid_spec,
        out_shape=jax.ShapeDtypeStruct((NPAD, DM), jnp.float32),
        compiler_params=pltpu.CompilerParams(
            dimension_semantics=("arbitrary", "arbitrary"),
        ),
    )(block_pol, x_sorted, W1, W2, b1r, b2)


def _route(pi, P, N, NB):
    TM = _TM
    NPAD = NB * TM
    pids = jnp.arange(P, dtype=jnp.int32)
    oh = (pi[:, None] == pids[None, :]).astype(jnp.int32)
    cs = jnp.cumsum(oh, axis=0)
    counts = cs[-1]
    rank = jnp.sum((cs - oh) * oh, axis=1)
    nblk = (counts + TM - 1) // TM
    bsum = jnp.cumsum(nblk)
    blk_start = jnp.concatenate([jnp.zeros((1,), jnp.int32),
                                 bsum[:-1].astype(jnp.int32)])
    padded_start = (blk_start * TM).astype(jnp.int32)

    pos = (jnp.sum(oh * padded_start[None, :], axis=1) + rank).astype(jnp.int32)

    r = jnp.arange(NPAD, dtype=jnp.int32)
    row_src = (r % N).at[pos].set(jnp.arange(N, dtype=jnp.int32))

    bstart = jnp.arange(NB, dtype=jnp.int32) * TM
    block_pol = (jnp.sum((padded_start[None, :] <= bstart[:, None])
                         .astype(jnp.int32), axis=1) - 1).astype(jnp.int32)
    return row_src, block_pol, pos


def kernel(latents, policy_indices, actions, W1, b1, W2, b2):
    N, DM = latents.shape
    DA = actions.shape[1]
    P, DK, DF = W1.shape
    FB = DF // _NFF

    NB = N // _TM + P
    while (NB * _TM) % (8 * _NW) != 0:
        NB += 1

    pi = policy_indices.astype(jnp.int32)
    row_src, block_pol, pos = _route(pi, P, N, NB)

    a_pad = jnp.pad(actions, ((0, 0), (128 - DA, 0)))
    x_lin = _tc_prep(latents, a_pad)
    x_sorted = _sc_gather_rows(x_lin, row_src, chunk=40)

    b1r = b1.reshape(P, _NFF, FB)
    W1b = W1.astype(jnp.bfloat16)
    W2b = W2.astype(jnp.bfloat16)
    y_sorted = _tc_grouped_mlp(block_pol, x_sorted, W1b, b1r, W2b, b2)

    out = _sc_gather_rows(y_sorted, pos, chunk=32)
    return out
